# baseline (speedup 1.0000x reference)
import functools

import jax
import jax.numpy as jnp
from jax import lax
from jax.experimental import pallas as pl
from jax.experimental.pallas import tpu as pltpu
from jax.experimental.pallas import tpu_sc as plsc

NC = 2
NS = 16
NW = NC * NS
L = 16
CHUNK = 128


TCOLS = 2048


@functools.partial(jax.jit, static_argnames=("V", "D"))
def _tc_repack(table_t, sel, *, V, D):
    n_units = TCOLS // 128
    rows_per_block = TCOLS // 4
    grid = (V + TCOLS - 1) // TCOLS

    def body(in_ref, sel_ref, out_ref):
        k = pl.program_id(0)

        def unit(t, mask):
            u = in_ref[:, t * 128:(t + 1) * 128]
            if mask:
                col = jax.lax.broadcasted_iota(jnp.int32, (D, 128), 1)
                u = jnp.where(col < (V - k * TCOLS - t * 128), u, 0.0)
            blks = [
                jax.lax.dot_general(
                    sel_ref[a], u,
                    (((0,), (1,)), ((), ())),
                    preferred_element_type=jnp.float32,
                )
                for a in range(4)
            ]
            out_ref[t * 32 + 0:t * 32 + 32, :] = jnp.concatenate(blks, axis=1)

        @pl.when(k < grid - 1)
        def _():
            for t in range(n_units):
                unit(t, False)

        @pl.when(k == grid - 1)
        def _():
            for t in range(n_units):
                unit(t, True)

    return pl.pallas_call(
        body,
        grid=(grid,),
        in_specs=[
            pl.BlockSpec((D, TCOLS), lambda k: (0, k)),
            pl.BlockSpec((4, 128, 32), lambda k: (0, 0, 0)),
        ],
        out_specs=pl.BlockSpec((rows_per_block, 4 * D), lambda k: (k, 0)),
        out_shape=jax.ShapeDtypeStruct((V // 4, 4 * D), jnp.float32),
    )(table_t, sel)


@functools.partial(jax.jit, static_argnames=("B", "D", "K"))
def _sc_scores(h_t, table4, idx_g, *, B, D, K):
    n_per_w = B // NW
    n_chunks = (n_per_w * K) // CHUNK
    qs = n_per_w // CHUNK
    mesh = plsc.VectorSubcoreMesh(core_axis_name="c", subcore_axis_name="s")

    def body(ht_hbm, table_hbm, idx_hbm, out_hbm,
             idx_v, idx4_v, ht_v, buf_a, buf_b, out_v, sem_a, sem_b, sem_h):
        cid = lax.axis_index("c")
        sid = lax.axis_index("s")
        wid = sid * NC + cid
        base = wid * n_per_w

        pltpu.sync_copy(idx_hbm.at[wid], idx_v)
        hcp = pltpu.async_copy(ht_hbm.at[:, pl.ds(base, n_per_w)], ht_v, sem_h)

        for r in range(n_chunks):
            for c8 in range(CHUNK // L):
                v = idx_v[r, pl.ds(c8 * L, L)]
                idx4_v[r, pl.ds(c8 * L, L)] = lax.shift_right_logical(v, 2)

        def fire(k, buf, sem):
            pltpu.async_copy(table_hbm.at[idx4_v.at[k]], buf, sem)

        def drain(buf, sem):
            pltpu.make_async_copy(table_hbm.at[idx4_v.at[0]], buf, sem).wait()

        fire(0, buf_a, sem_a)
        fire(1, buf_b, sem_b)
        hcp.wait()

        lanes = lax.iota(jnp.int32, L)

        def compute_chunk(k, buf):
            q = lax.rem(k, qs)
            for g in range(CHUNK // L):
                rem = idx_v[k, pl.ds(g * L, L)] & 3
                colbase = rem * D
                rows16 = g * L + lanes
                acc = jnp.zeros((L,), jnp.float32)
                for d in range(D):
                    e = plsc.load_gather(buf, [rows16, colbase + d])
                    hc = ht_v[d, pl.ds(q * CHUNK + g * L, L)]
                    acc = acc + e * hc
                z = jnp.exp(-jnp.abs(acc))
                s = jnp.where(acc >= 0, 1.0 / (1.0 + z), z / (1.0 + z))
                out_v[pl.ds(k * CHUNK + g * L, L)] = s

        def step(i, carry):
            k0 = 2 * i
            k1 = 2 * i + 1
            drain(buf_a, sem_a)
            compute_chunk(k0, buf_a)
            nxt0 = k0 + 2

            @pl.when(nxt0 < n_chunks)
            def _():
                fire(nxt0, buf_a, sem_a)

            drain(buf_b, sem_b)
            compute_chunk(k1, buf_b)
            nxt1 = k1 + 2

            @pl.when(nxt1 < n_chunks)
            def _():
                fire(nxt1, buf_b, sem_b)

            return carry

        lax.fori_loop(0, n_chunks // 2, step, 0)

        for j in range(K):
            pltpu.sync_copy(out_v.at[pl.ds(j * n_per_w, n_per_w)],
                            out_hbm.at[j, pl.ds(base, n_per_w)])

    return pl.kernel(
        body,
        out_type=jax.ShapeDtypeStruct((K, B), jnp.float32),
        mesh=mesh,
        compiler_params=pltpu.CompilerParams(
            needs_layout_passes=False, use_tc_tiling_on_sc=False
        ),
        scratch_types=[
            pltpu.VMEM((n_chunks, CHUNK), jnp.int32),
            pltpu.VMEM((n_chunks, CHUNK), jnp.int32),
            pltpu.VMEM((D, n_per_w), jnp.float32),
            pltpu.VMEM((CHUNK, CHUNK), jnp.float32),
            pltpu.VMEM((CHUNK, CHUNK), jnp.float32),
            pltpu.VMEM((n_per_w * K,), jnp.float32),
            pltpu.SemaphoreType.DMA,
            pltpu.SemaphoreType.DMA,
            pltpu.SemaphoreType.DMA,
        ],
    )(h_t, table4, idx_g)


def kernel(h, target_index, table, negative_sample):
    B, D = h.shape
    V = table.shape[0]
    S = negative_sample.shape[1]
    K = S + 1
    n_per_w = B // NW
    idx_t = jnp.concatenate([target_index[None, :], negative_sample.T], axis=0)
    idx_g = (idx_t.reshape(K, NW, n_per_w // CHUNK, CHUNK)
             .transpose(1, 0, 2, 3)
             .reshape(NW, (K * n_per_w) // CHUNK, CHUNK))
    sel = (jnp.arange(128, dtype=jnp.int32)[None, :, None]
           == (4 * jnp.arange(32, dtype=jnp.int32)[None, None, :]
               + jnp.arange(4, dtype=jnp.int32)[:, None, None])
           ).astype(jnp.float32)
    table4 = _tc_repack(table.T, sel, V=V, D=D)
    scores = _sc_scores(h.T, table4, idx_g, B=B, D=D, K=K)
    pos_out = scores[0][:, None]
    neg_out = scores[1:].T
    pos_label = jnp.ones((B, 1), dtype=jnp.float32)
    neg_label = jnp.zeros((B, S), dtype=jnp.float32)
    return (pos_out, pos_label, neg_out, neg_label)


# speedup vs baseline: 1.1823x; 1.1823x over previous
import functools

import jax
import jax.numpy as jnp
from jax import lax
from jax.experimental import pallas as pl
from jax.experimental.pallas import tpu as pltpu
from jax.experimental.pallas import tpu_sc as plsc

NC = 2
NS = 16
NW = NC * NS
L = 16
CHUNK = 128


TCOLS = 8192


@functools.partial(jax.jit, static_argnames=("V", "D"))
def _tc_repack(table_t, sel, *, V, D):
    n_units = TCOLS // 128
    rows_per_block = TCOLS // 4
    grid = (V + TCOLS - 1) // TCOLS

    def body(in_ref, sel_ref, out_ref):
        k = pl.program_id(0)

        def unit(t, mask):
            u = in_ref[:, t * 128:(t + 1) * 128]
            if mask:
                col = jax.lax.broadcasted_iota(jnp.int32, (D, 128), 1)
                u = jnp.where(col < (V - k * TCOLS - t * 128), u, 0.0)
            blks = [
                jax.lax.dot_general(
                    sel_ref[a], u,
                    (((0,), (1,)), ((), ())),
                    preferred_element_type=jnp.float32,
                )
                for a in range(4)
            ]
            out_ref[t * 32 + 0:t * 32 + 32, :] = jnp.concatenate(blks, axis=1)

        @pl.when(k < grid - 1)
        def _():
            for t in range(n_units):
                unit(t, False)

        @pl.when(k == grid - 1)
        def _():
            for t in range(n_units):
                unit(t, True)

    return pl.pallas_call(
        body,
        grid=(grid,),
        in_specs=[
            pl.BlockSpec((D, TCOLS), lambda k: (0, k)),
            pl.BlockSpec((4, 128, 32), lambda k: (0, 0, 0)),
        ],
        out_specs=pl.BlockSpec((rows_per_block, 4 * D), lambda k: (k, 0)),
        out_shape=jax.ShapeDtypeStruct((V // 4, 4 * D), jnp.float32),
    )(table_t, sel)


@functools.partial(jax.jit, static_argnames=("B", "D", "K"))
def _sc_scores(h_t, table4, idx_g, *, B, D, K):
    n_per_w = B // NW
    n_chunks = (n_per_w * K) // CHUNK
    qs = n_per_w // CHUNK
    mesh = plsc.VectorSubcoreMesh(core_axis_name="c", subcore_axis_name="s")

    def body(ht_hbm, table_hbm, idx_hbm, out_hbm,
             idx_v, idx4_v, ht_v, buf_a, buf_b, out_v, sem_a, sem_b, sem_h):
        cid = lax.axis_index("c")
        sid = lax.axis_index("s")
        wid = sid * NC + cid
        base = wid * n_per_w

        pltpu.sync_copy(idx_hbm.at[wid], idx_v)
        hcp = pltpu.async_copy(ht_hbm.at[:, pl.ds(base, n_per_w)], ht_v, sem_h)

        for r in range(n_chunks):
            for c8 in range(CHUNK // L):
                v = idx_v[r, pl.ds(c8 * L, L)]
                idx4_v[r, pl.ds(c8 * L, L)] = lax.shift_right_logical(v, 2)

        def fire(k, buf, sem):
            pltpu.async_copy(table_hbm.at[idx4_v.at[k]], buf, sem)

        def drain(buf, sem):
            pltpu.make_async_copy(table_hbm.at[idx4_v.at[0]], buf, sem).wait()

        fire(0, buf_a, sem_a)
        fire(1, buf_b, sem_b)
        hcp.wait()

        lanes = lax.iota(jnp.int32, L)

        def compute_chunk(k, buf):
            q = lax.rem(k, qs)
            for g in range(CHUNK // L):
                rem = idx_v[k, pl.ds(g * L, L)] & 3
                colbase = rem * D
                rows16 = g * L + lanes
                acc = jnp.zeros((L,), jnp.float32)
                for d in range(D):
                    e = plsc.load_gather(buf, [rows16, colbase + d])
                    hc = ht_v[d, pl.ds(q * CHUNK + g * L, L)]
                    acc = acc + e * hc
                z = jnp.exp(-jnp.abs(acc))
                s = jnp.where(acc >= 0, 1.0 / (1.0 + z), z / (1.0 + z))
                out_v[pl.ds(k * CHUNK + g * L, L)] = s

        def step(i, carry):
            k0 = 2 * i
            k1 = 2 * i + 1
            drain(buf_a, sem_a)
            compute_chunk(k0, buf_a)
            nxt0 = k0 + 2

            @pl.when(nxt0 < n_chunks)
            def _():
                fire(nxt0, buf_a, sem_a)

            drain(buf_b, sem_b)
            compute_chunk(k1, buf_b)
            nxt1 = k1 + 2

            @pl.when(nxt1 < n_chunks)
            def _():
                fire(nxt1, buf_b, sem_b)

            return carry

        lax.fori_loop(0, n_chunks // 2, step, 0)

        for j in range(K):
            pltpu.sync_copy(out_v.at[pl.ds(j * n_per_w, n_per_w)],
                            out_hbm.at[j, pl.ds(base, n_per_w)])

    return pl.kernel(
        body,
        out_type=jax.ShapeDtypeStruct((K, B), jnp.float32),
        mesh=mesh,
        compiler_params=pltpu.CompilerParams(
            needs_layout_passes=False, use_tc_tiling_on_sc=False
        ),
        scratch_types=[
            pltpu.VMEM((n_chunks, CHUNK), jnp.int32),
            pltpu.VMEM((n_chunks, CHUNK), jnp.int32),
            pltpu.VMEM((D, n_per_w), jnp.float32),
            pltpu.VMEM((CHUNK, CHUNK), jnp.float32),
            pltpu.VMEM((CHUNK, CHUNK), jnp.float32),
            pltpu.VMEM((n_per_w * K,), jnp.float32),
            pltpu.SemaphoreType.DMA,
            pltpu.SemaphoreType.DMA,
            pltpu.SemaphoreType.DMA,
        ],
    )(h_t, table4, idx_g)


def kernel(h, target_index, table, negative_sample):
    B, D = h.shape
    V = table.shape[0]
    S = negative_sample.shape[1]
    K = S + 1
    n_per_w = B // NW
    idx_t = jnp.concatenate([target_index[None, :], negative_sample.T], axis=0)
    idx_g = (idx_t.reshape(K, NW, n_per_w // CHUNK, CHUNK)
             .transpose(1, 0, 2, 3)
             .reshape(NW, (K * n_per_w) // CHUNK, CHUNK))
    sel = (jnp.arange(128, dtype=jnp.int32)[None, :, None]
           == (4 * jnp.arange(32, dtype=jnp.int32)[None, None, :]
               + jnp.arange(4, dtype=jnp.int32)[:, None, None])
           ).astype(jnp.float32)
    table4 = _tc_repack(table.T, sel, V=V, D=D)
    scores = _sc_scores(h.T, table4, idx_g, B=B, D=D, K=K)
    pos_out = scores[0][:, None]
    neg_out = scores[1:].T
    pos_label = jnp.ones((B, 1), dtype=jnp.float32)
    neg_label = jnp.zeros((B, S), dtype=jnp.float32)
    return (pos_out, pos_label, neg_out, neg_label)


# speedup vs baseline: 1.1911x; 1.0074x over previous
import functools

import jax
import jax.numpy as jnp
from jax import lax
from jax.experimental import pallas as pl
from jax.experimental.pallas import tpu as pltpu
from jax.experimental.pallas import tpu_sc as plsc

NC = 2
NS = 16
NW = NC * NS
L = 16
CHUNK = 128


TCOLS = 32768


@functools.partial(jax.jit, static_argnames=("V", "D"))
def _tc_repack(table_t, sel, *, V, D):
    n_units = TCOLS // 128
    rows_per_block = TCOLS // 4
    grid = (V + TCOLS - 1) // TCOLS

    def body(in_ref, sel_ref, out_ref):
        k = pl.program_id(0)

        def unit(t, mask):
            u = in_ref[:, t * 128:(t + 1) * 128]
            if mask:
                col = jax.lax.broadcasted_iota(jnp.int32, (D, 128), 1)
                u = jnp.where(col < (V - k * TCOLS - t * 128), u, 0.0)
            blks = [
                jax.lax.dot_general(
                    sel_ref[a], u,
                    (((0,), (1,)), ((), ())),
                    preferred_element_type=jnp.float32,
                )
                for a in range(4)
            ]
            out_ref[t * 32 + 0:t * 32 + 32, :] = jnp.concatenate(blks, axis=1)

        @pl.when(k < grid - 1)
        def _():
            for t in range(n_units):
                unit(t, False)

        @pl.when(k == grid - 1)
        def _():
            for t in range(n_units):
                unit(t, True)

    return pl.pallas_call(
        body,
        grid=(grid,),
        in_specs=[
            pl.BlockSpec((D, TCOLS), lambda k: (0, k)),
            pl.BlockSpec((4, 128, 32), lambda k: (0, 0, 0)),
        ],
        out_specs=pl.BlockSpec((rows_per_block, 4 * D), lambda k: (k, 0)),
        out_shape=jax.ShapeDtypeStruct((V // 4, 4 * D), jnp.float32),
    )(table_t, sel)


@functools.partial(jax.jit, static_argnames=("B", "D", "K"))
def _sc_scores(h_t, table4, idx_g, *, B, D, K):
    n_per_w = B // NW
    n_chunks = (n_per_w * K) // CHUNK
    qs = n_per_w // CHUNK
    mesh = plsc.VectorSubcoreMesh(core_axis_name="c", subcore_axis_name="s")

    def body(ht_hbm, table_hbm, idx_hbm, out_hbm,
             idx_v, idx4_v, ht_v, buf_a, buf_b, out_v, sem_a, sem_b, sem_h):
        cid = lax.axis_index("c")
        sid = lax.axis_index("s")
        wid = sid * NC + cid
        base = wid * n_per_w

        pltpu.sync_copy(idx_hbm.at[wid], idx_v)
        hcp = pltpu.async_copy(ht_hbm.at[:, pl.ds(base, n_per_w)], ht_v, sem_h)

        for r in range(n_chunks):
            for c8 in range(CHUNK // L):
                v = idx_v[r, pl.ds(c8 * L, L)]
                idx4_v[r, pl.ds(c8 * L, L)] = lax.shift_right_logical(v, 2)

        def fire(k, buf, sem):
            pltpu.async_copy(table_hbm.at[idx4_v.at[k]], buf, sem)

        def drain(buf, sem):
            pltpu.make_async_copy(table_hbm.at[idx4_v.at[0]], buf, sem).wait()

        fire(0, buf_a, sem_a)
        fire(1, buf_b, sem_b)
        hcp.wait()

        lanes = lax.iota(jnp.int32, L)

        def compute_chunk(k, buf):
            q = lax.rem(k, qs)
            for g in range(CHUNK // L):
                rem = idx_v[k, pl.ds(g * L, L)] & 3
                colbase = rem * D
                rows16 = g * L + lanes
                acc = jnp.zeros((L,), jnp.float32)
                for d in range(D):
                    e = plsc.load_gather(buf, [rows16, colbase + d])
                    hc = ht_v[d, pl.ds(q * CHUNK + g * L, L)]
                    acc = acc + e * hc
                z = jnp.exp(-jnp.abs(acc))
                s = jnp.where(acc >= 0, 1.0 / (1.0 + z), z / (1.0 + z))
                out_v[pl.ds(k * CHUNK + g * L, L)] = s

        def step(i, carry):
            k0 = 2 * i
            k1 = 2 * i + 1
            drain(buf_a, sem_a)
            compute_chunk(k0, buf_a)
            nxt0 = k0 + 2

            @pl.when(nxt0 < n_chunks)
            def _():
                fire(nxt0, buf_a, sem_a)

            drain(buf_b, sem_b)
            compute_chunk(k1, buf_b)
            nxt1 = k1 + 2

            @pl.when(nxt1 < n_chunks)
            def _():
                fire(nxt1, buf_b, sem_b)

            return carry

        lax.fori_loop(0, n_chunks // 2, step, 0)

        for j in range(K):
            pltpu.sync_copy(out_v.at[pl.ds(j * n_per_w, n_per_w)],
                            out_hbm.at[j, pl.ds(base, n_per_w)])

    return pl.kernel(
        body,
        out_type=jax.ShapeDtypeStruct((K, B), jnp.float32),
        mesh=mesh,
        compiler_params=pltpu.CompilerParams(
            needs_layout_passes=False, use_tc_tiling_on_sc=False
        ),
        scratch_types=[
            pltpu.VMEM((n_chunks, CHUNK), jnp.int32),
            pltpu.VMEM((n_chunks, CHUNK), jnp.int32),
            pltpu.VMEM((D, n_per_w), jnp.float32),
            pltpu.VMEM((CHUNK, CHUNK), jnp.float32),
            pltpu.VMEM((CHUNK, CHUNK), jnp.float32),
            pltpu.VMEM((n_per_w * K,), jnp.float32),
            pltpu.SemaphoreType.DMA,
            pltpu.SemaphoreType.DMA,
            pltpu.SemaphoreType.DMA,
        ],
    )(h_t, table4, idx_g)


def kernel(h, target_index, table, negative_sample):
    B, D = h.shape
    V = table.shape[0]
    S = negative_sample.shape[1]
    K = S + 1
    n_per_w = B // NW
    idx_t = jnp.concatenate([target_index[None, :], negative_sample.T], axis=0)
    idx_g = (idx_t.reshape(K, NW, n_per_w // CHUNK, CHUNK)
             .transpose(1, 0, 2, 3)
             .reshape(NW, (K * n_per_w) // CHUNK, CHUNK))
    sel = (jnp.arange(128, dtype=jnp.int32)[None, :, None]
           == (4 * jnp.arange(32, dtype=jnp.int32)[None, None, :]
               + jnp.arange(4, dtype=jnp.int32)[:, None, None])
           ).astype(jnp.float32)
    table4 = _tc_repack(table.T, sel, V=V, D=D)
    scores = _sc_scores(h.T, table4, idx_g, B=B, D=D, K=K)
    pos_out = scores[0][:, None]
    neg_out = scores[1:].T
    pos_label = jnp.ones((B, 1), dtype=jnp.float32)
    neg_label = jnp.zeros((B, S), dtype=jnp.float32)
    return (pos_out, pos_label, neg_out, neg_label)


# speedup vs baseline: 1.2042x; 1.0110x over previous
import functools

import jax
import jax.numpy as jnp
from jax import lax
from jax.experimental import pallas as pl
from jax.experimental.pallas import tpu as pltpu
from jax.experimental.pallas import tpu_sc as plsc

NC = 2
NS = 16
NW = NC * NS
L = 16
CHUNK = 128


TCOLS = 32768


@functools.partial(jax.jit, static_argnames=("V", "D"))
def _tc_repack(table_t, sel, *, V, D):
    n_units = TCOLS // 128
    rows_per_block = TCOLS // 4
    grid = (V + TCOLS - 1) // TCOLS

    def body(in_ref, sel_ref, out_ref):
        k = pl.program_id(0)

        def unit(t, mask):
            u = in_ref[:, t * 128:(t + 1) * 128]
            if mask:
                col = jax.lax.broadcasted_iota(jnp.int32, (D, 128), 1)
                u = jnp.where(col < (V - k * TCOLS - t * 128), u, 0.0)
            u = u.astype(jnp.bfloat16)
            blks = [
                jax.lax.dot_general(
                    sel_ref[a], u,
                    (((0,), (1,)), ((), ())),
                    preferred_element_type=jnp.float32,
                )
                for a in range(4)
            ]
            out_ref[t * 32 + 0:t * 32 + 32, :] = jnp.concatenate(blks, axis=1)

        @pl.when(k < grid - 1)
        def _():
            for t in range(n_units):
                unit(t, False)

        @pl.when(k == grid - 1)
        def _():
            for t in range(n_units):
                unit(t, True)

    return pl.pallas_call(
        body,
        grid=(grid,),
        in_specs=[
            pl.BlockSpec((D, TCOLS), lambda k: (0, k)),
            pl.BlockSpec((4, 128, 32), lambda k: (0, 0, 0)),
        ],
        out_specs=pl.BlockSpec((rows_per_block, 4 * D), lambda k: (k, 0)),
        out_shape=jax.ShapeDtypeStruct((V // 4, 4 * D), jnp.float32),
    )(table_t, sel)


@functools.partial(jax.jit, static_argnames=("B", "D", "K"))
def _sc_scores(h_t, table4, idx_g, *, B, D, K):
    n_per_w = B // NW
    n_chunks = (n_per_w * K) // CHUNK
    qs = n_per_w // CHUNK
    mesh = plsc.VectorSubcoreMesh(core_axis_name="c", subcore_axis_name="s")

    def body(ht_hbm, table_hbm, idx_hbm, out_hbm,
             idx_v, idx4_v, ht_v, buf_a, buf_b, out_v, sem_a, sem_b, sem_h):
        cid = lax.axis_index("c")
        sid = lax.axis_index("s")
        wid = sid * NC + cid
        base = wid * n_per_w

        pltpu.sync_copy(idx_hbm.at[wid], idx_v)
        hcp = pltpu.async_copy(ht_hbm.at[:, pl.ds(base, n_per_w)], ht_v, sem_h)

        for r in range(n_chunks):
            for c8 in range(CHUNK // L):
                v = idx_v[r, pl.ds(c8 * L, L)]
                idx4_v[r, pl.ds(c8 * L, L)] = lax.shift_right_logical(v, 2)

        def fire(k, buf, sem):
            pltpu.async_copy(table_hbm.at[idx4_v.at[k]], buf, sem)

        def drain(buf, sem):
            pltpu.make_async_copy(table_hbm.at[idx4_v.at[0]], buf, sem).wait()

        fire(0, buf_a, sem_a)
        fire(1, buf_b, sem_b)
        hcp.wait()

        lanes = lax.iota(jnp.int32, L)

        def compute_chunk(k, buf):
            q = lax.rem(k, qs)
            for g in range(CHUNK // L):
                rem = idx_v[k, pl.ds(g * L, L)] & 3
                colbase = rem * D
                rows16 = g * L + lanes
                acc = jnp.zeros((L,), jnp.float32)
                for d in range(D):
                    e = plsc.load_gather(buf, [rows16, colbase + d])
                    hc = ht_v[d, pl.ds(q * CHUNK + g * L, L)]
                    acc = acc + e * hc
                z = jnp.exp(-jnp.abs(acc))
                s = jnp.where(acc >= 0, 1.0 / (1.0 + z), z / (1.0 + z))
                out_v[pl.ds(k * CHUNK + g * L, L)] = s

        def step(i, carry):
            k0 = 2 * i
            k1 = 2 * i + 1
            drain(buf_a, sem_a)
            compute_chunk(k0, buf_a)
            nxt0 = k0 + 2

            @pl.when(nxt0 < n_chunks)
            def _():
                fire(nxt0, buf_a, sem_a)

            drain(buf_b, sem_b)
            compute_chunk(k1, buf_b)
            nxt1 = k1 + 2

            @pl.when(nxt1 < n_chunks)
            def _():
                fire(nxt1, buf_b, sem_b)

            return carry

        lax.fori_loop(0, n_chunks // 2, step, 0)

        for j in range(K):
            pltpu.sync_copy(out_v.at[pl.ds(j * n_per_w, n_per_w)],
                            out_hbm.at[j, pl.ds(base, n_per_w)])

    return pl.kernel(
        body,
        out_type=jax.ShapeDtypeStruct((K, B), jnp.float32),
        mesh=mesh,
        compiler_params=pltpu.CompilerParams(
            needs_layout_passes=False, use_tc_tiling_on_sc=False
        ),
        scratch_types=[
            pltpu.VMEM((n_chunks, CHUNK), jnp.int32),
            pltpu.VMEM((n_chunks, CHUNK), jnp.int32),
            pltpu.VMEM((D, n_per_w), jnp.float32),
            pltpu.VMEM((CHUNK, CHUNK), jnp.float32),
            pltpu.VMEM((CHUNK, CHUNK), jnp.float32),
            pltpu.VMEM((n_per_w * K,), jnp.float32),
            pltpu.SemaphoreType.DMA,
            pltpu.SemaphoreType.DMA,
            pltpu.SemaphoreType.DMA,
        ],
    )(h_t, table4, idx_g)


def kernel(h, target_index, table, negative_sample):
    B, D = h.shape
    V = table.shape[0]
    S = negative_sample.shape[1]
    K = S + 1
    n_per_w = B // NW
    idx_t = jnp.concatenate([target_index[None, :], negative_sample.T], axis=0)
    idx_g = (idx_t.reshape(K, NW, n_per_w // CHUNK, CHUNK)
             .transpose(1, 0, 2, 3)
             .reshape(NW, (K * n_per_w) // CHUNK, CHUNK))
    sel = (jnp.arange(128, dtype=jnp.int32)[None, :, None]
           == (4 * jnp.arange(32, dtype=jnp.int32)[None, None, :]
               + jnp.arange(4, dtype=jnp.int32)[:, None, None])
           ).astype(jnp.bfloat16)
    table4 = _tc_repack(table.T, sel, V=V, D=D)
    scores = _sc_scores(h.T, table4, idx_g, B=B, D=D, K=K)
    pos_out = scores[0][:, None]
    neg_out = scores[1:].T
    pos_label = jnp.ones((B, 1), dtype=jnp.float32)
    neg_label = jnp.zeros((B, S), dtype=jnp.float32)
    return (pos_out, pos_label, neg_out, neg_label)
